# baseline (speedup 1.0000x reference)
import functools

import jax
import jax.numpy as jnp
from jax import lax
from jax.experimental import pallas as pl
from jax.experimental.pallas import tpu as pltpu
from jax.experimental.pallas import tpu_sc as plsc

B = 16384
D = 64
NC = 2
NS = 16
NW = NC * NS
BPW = B // NW
CHUNK = 128
NCHUNK = BPW // CHUNK
VROW = 500000


def _sc_body(uidx_hbm, pidx_hbm, uw_hbm, pw_hbm, out_hbm,
             uidx_v, pidx_v, urow_v, prow_v, ubuf_v, pbuf_v, out_v, sems):
    wid = lax.axis_index("s") * NC + lax.axis_index("c")
    base = wid * BPW
    lane = lax.iota(jnp.int32, 16)

    pltpu.sync_copy(uidx_hbm.at[pl.ds(wid * NCHUNK, NCHUNK)], uidx_v)
    pltpu.sync_copy(pidx_hbm.at[pl.ds(wid * NCHUNK, NCHUNK)], pidx_v)

    for c in range(NCHUNK):
        for g in range(CHUNK // 16):
            sl = pl.ds(g * 16, 16)
            urow_v[c, sl] = lax.shift_right_logical(uidx_v[c, sl], 1)
            prow_v[c, sl] = lax.shift_right_logical(pidx_v[c, sl], 1)

    def fire(c):
        ring = c % 2
        return (pltpu.async_copy(uw_hbm.at[urow_v.at[c]],
                                 ubuf_v.at[ring], sems.at[ring]),
                pltpu.async_copy(pw_hbm.at[prow_v.at[c]],
                                 pbuf_v.at[ring], sems.at[ring]))

    def compute(c):
        ring = c % 2
        ub = ubuf_v.at[ring]
        pb = pbuf_v.at[ring]

        def g_body(g, carry):
            sl = pl.ds(g * 16, 16)
            ucol = (uidx_v[c, sl] & 1) * D
            pcol = (pidx_v[c, sl] & 1) * D
            row16 = g * 16 + lane
            acc = jnp.zeros((16,), jnp.float32)
            for d in range(D):
                uvals = plsc.load_gather(ub, [row16, ucol + d])
                pvals = plsc.load_gather(pb, [row16, pcol + d])
                acc = acc + uvals * pvals
            out_v[pl.ds(c * CHUNK + g * 16, 16)] = acc
            return carry

        lax.fori_loop(0, CHUNK // 16, g_body, 0)

    inflight = [fire(0), fire(1)]
    for c in range(NCHUNK):
        for cp in inflight.pop(0):
            cp.wait()
        compute(c)
        if c + 2 < NCHUNK:
            inflight.append(fire(c + 2))

    pltpu.sync_copy(out_v, out_hbm.at[pl.ds(base, BPW)])


@jax.jit
def _collab_dot(uidx, pidx, users_r, products_r):
    run = functools.partial(
        pl.kernel,
        mesh=plsc.VectorSubcoreMesh(core_axis_name="c", subcore_axis_name="s"),
        compiler_params=pltpu.CompilerParams(needs_layout_passes=False),
        out_type=jax.ShapeDtypeStruct((B,), jnp.float32),
        scratch_types=[
            pltpu.VMEM((NCHUNK, CHUNK), jnp.int32),
            pltpu.VMEM((NCHUNK, CHUNK), jnp.int32),
            pltpu.VMEM((NCHUNK, CHUNK), jnp.int32),
            pltpu.VMEM((NCHUNK, CHUNK), jnp.int32),
            pltpu.VMEM((2, CHUNK, 128), jnp.float32),
            pltpu.VMEM((2, CHUNK, 128), jnp.float32),
            pltpu.VMEM((BPW,), jnp.float32),
            pltpu.SemaphoreType.DMA((2,)),
        ],
    )(_sc_body)
    return run(uidx, pidx, users_r, products_r)


def kernel(inputs, users_w, products_w):
    uidx = inputs[:, 0].reshape(B // CHUNK, CHUNK)
    pidx = inputs[:, 1].reshape(B // CHUNK, CHUNK)
    users_r = users_w.reshape(VROW, 128)
    products_r = products_w.reshape(VROW, 128)
    out = _collab_dot(uidx, pidx, users_r, products_r)
    return out[:, None]


# speedup vs baseline: 1.0937x; 1.0937x over previous
import functools

import jax
import jax.numpy as jnp
from jax import lax
from jax.experimental import pallas as pl
from jax.experimental.pallas import tpu as pltpu
from jax.experimental.pallas import tpu_sc as plsc

B = 16384
D = 64
NC = 2
NS = 16
NW = NC * NS
BPW = B // NW
CHUNK = 128
NCHUNK = BPW // CHUNK

_GATHER_DNUMS = lax.GatherDimensionNumbers(
    offset_dims=(), collapsed_slice_dims=(0,), start_index_map=(0,))


def _shuffle(x, idx):
    return lax.gather(x, idx[:, None], _GATHER_DNUMS, slice_sizes=(1,),
                      mode=lax.GatherScatterMode.PROMISE_IN_BOUNDS)


def _sc_body(uidx_hbm, pidx_hbm, uw_hbm, pw_hbm, out_hbm,
             uidx_v, pidx_v, ubuf_v, pbuf_v, out_v, sems):
    wid = lax.axis_index("s") * NC + lax.axis_index("c")
    base = wid * BPW
    lane = lax.iota(jnp.int32, 16)

    pltpu.sync_copy(uidx_hbm.at[pl.ds(wid * NCHUNK, NCHUNK)], uidx_v)
    pltpu.sync_copy(pidx_hbm.at[pl.ds(wid * NCHUNK, NCHUNK)], pidx_v)

    def fire(c):
        ring = c % 2
        return (pltpu.async_copy(uw_hbm.at[uidx_v.at[c]],
                                 ubuf_v.at[ring], sems.at[ring]),
                pltpu.async_copy(pw_hbm.at[pidx_v.at[c]],
                                 pbuf_v.at[ring], sems.at[ring]))

    def compute(c):
        ring = c % 2
        ub = ubuf_v.at[ring]
        pb = pbuf_v.at[ring]

        def g_body(g, carry):
            out_vec = jnp.zeros((16,), jnp.float32)
            for r in range(16):
                row = g * 16 + r
                acc = ub[row, pl.ds(0, 16)] * pb[row, pl.ds(0, 16)]
                for dd in range(1, D // 16):
                    acc = acc + (ub[row, pl.ds(dd * 16, 16)]
                                 * pb[row, pl.ds(dd * 16, 16)])
                for sh in (8, 4, 2, 1):
                    acc = acc + _shuffle(acc, lane ^ sh)
                out_vec = jnp.where(lane == r, acc, out_vec)
            out_v[pl.ds(c * CHUNK + g * 16, 16)] = out_vec
            return carry

        lax.fori_loop(0, CHUNK // 16, g_body, 0)

    inflight = [fire(0), fire(1)]
    for c in range(NCHUNK):
        for cp in inflight.pop(0):
            cp.wait()
        compute(c)
        if c + 2 < NCHUNK:
            inflight.append(fire(c + 2))

    pltpu.sync_copy(out_v, out_hbm.at[pl.ds(base, BPW)])


@jax.jit
def _collab_dot(uidx, pidx, users_p, products_p):
    run = functools.partial(
        pl.kernel,
        mesh=plsc.VectorSubcoreMesh(core_axis_name="c", subcore_axis_name="s"),
        compiler_params=pltpu.CompilerParams(needs_layout_passes=False),
        out_type=jax.ShapeDtypeStruct((B,), jnp.float32),
        scratch_types=[
            pltpu.VMEM((NCHUNK, CHUNK), jnp.int32),
            pltpu.VMEM((NCHUNK, CHUNK), jnp.int32),
            pltpu.VMEM((2, CHUNK, 128), jnp.float32),
            pltpu.VMEM((2, CHUNK, 128), jnp.float32),
            pltpu.VMEM((BPW,), jnp.float32),
            pltpu.SemaphoreType.DMA((2,)),
        ],
    )(_sc_body)
    return run(uidx, pidx, users_p, products_p)


def kernel(inputs, users_w, products_w):
    uidx = inputs[:, 0].reshape(B // CHUNK, CHUNK)
    pidx = inputs[:, 1].reshape(B // CHUNK, CHUNK)
    users_p = jnp.pad(users_w, ((0, 0), (0, 128 - D)))
    products_p = jnp.pad(products_w, ((0, 0), (0, 128 - D)))
    out = _collab_dot(uidx, pidx, users_p, products_p)
    return out[:, None]
